# baseline (speedup 1.0000x reference)
import math

import jax
import jax.numpy as jnp
from jax import lax
from jax.experimental import pallas as pl
from jax.experimental.pallas import tpu as pltpu
from jax.experimental.pallas import tpu_sc as plsc

D = 64
SEQ = 200
NW = 32
L = 16
NK = D // L
SCALE = math.sqrt(D)
NG = 13


def _positional_encoding(d_model: int, length: int) -> jax.Array:
    position = jnp.arange(length, dtype=jnp.float32)[:, None]
    div_term = jnp.exp(
        jnp.arange(0, d_model, 2, dtype=jnp.float32)
        * (-(math.log(10000.0) / d_model))
    )
    pe = jnp.zeros((length, d_model), dtype=jnp.float32)
    pe = pe.at[:, 0::2].set(jnp.sin(position * div_term))
    pe = pe.at[:, 1::2].set(jnp.cos(position * div_term))
    return pe


def _rsqrt(v):
    i = lax.bitcast_convert_type(v, jnp.int32)
    i = 0x5F3759DF - lax.shift_right_logical(i, 1)
    y = lax.bitcast_convert_type(i, jnp.float32)
    for _ in range(2):
        y = y * (1.5 - 0.5 * v * y * y)
    return y


def _recip(v):
    i = lax.bitcast_convert_type(v, jnp.int32)
    i = 0x7EF311C3 - i
    y = lax.bitcast_convert_type(i, jnp.float32)
    for _ in range(2):
        y = y * (2.0 - v * y)
    return y


def _make_sc_call(n_rows: int):
    assert n_rows % (NW * SEQ) == 0
    rows_per_w = n_rows // NW
    seqs_per_w = rows_per_w // SEQ

    mesh = plsc.VectorSubcoreMesh(core_axis_name="c", subcore_axis_name="s")

    def body(x_hbm, table_hbm, pe_hbm, w_hbm, b_hbm, out_hbm,
             idx_v, pe_v, w_v, b_v, rows_a, rows_b, out_p,
             fidx_a, fidx_b, sem_ga, sem_gb, sem_s):
        cid = lax.axis_index("c")
        sid = lax.axis_index("s")
        wid = sid * 2 + cid
        base = wid * rows_per_w

        pltpu.sync_copy(x_hbm.at[pl.ds(base, rows_per_w)],
                        idx_v.at[pl.ds(0, rows_per_w)])
        idx_v[pl.ds(rows_per_w, L)] = jnp.zeros((L,), jnp.int32)
        pltpu.sync_copy(pe_hbm, pe_v)
        pltpu.sync_copy(w_hbm, w_v)
        pltpu.sync_copy(b_hbm, b_v)

        wv = [w_v[pl.ds(k * L, L)] for k in range(NK)]
        bv = [b_v[pl.ds(k * L, L)] for k in range(NK)]

        def gather(c, fidx_buf, rows_buf, sem):
            for i in range(NG):
                tv = idx_v[pl.ds(c * SEQ + i * L, L)]
                fidx_buf[pl.ds(i * L, L)] = lax.shift_right_logical(tv, 1)
            return pltpu.make_async_copy(
                table_hbm.at[fidx_buf.at[pl.ds(0, SEQ)]], rows_buf, sem)

        def wait_gather(c, fidx_buf, rows_buf, sem):
            return pltpu.make_async_copy(
                table_hbm.at[fidx_buf.at[pl.ds(0, SEQ)]], rows_buf, sem)

        def store(jpair, sem):
            fbase = pl.multiple_of((base + jpair * 2 * SEQ) // 2, 8)
            return pltpu.make_async_copy(
                out_p, out_hbm.at[pl.ds(fbase, SEQ)], sem)

        def compute(c, rows_buf, out_buf, out_off):
            @plsc.parallel_loop(0, NG, unroll=1)
            def group_body(i):
                g = jnp.minimum(i * L, SEQ - L)
                gh = lax.shift_right_logical(g, 1)
                xv = idx_v[pl.ds(c * SEQ + g, L)]
                for j in range(L):
                    r = g + j
                    fr = gh + (j >> 1)
                    ofr = out_off + fr
                    x_r = xv[j]
                    colb = lax.shift_left(x_r & 1, 6)
                    h = []
                    for k in range(NK):
                        e = rows_buf[r, pl.ds(colb + k * L, L)]
                        p = pe_v[fr, pl.ds((j & 1) * D + k * L, L)]
                        h.append(e * SCALE + p)
                    t = (h[0] + h[1]) + (h[2] + h[3])
                    s = jnp.sum(t)
                    qv = (h[0] * h[0] + h[1] * h[1]) \
                        + (h[2] * h[2] + h[3] * h[3])
                    q = jnp.sum(qv)
                    mean = s * (1.0 / D)
                    var = (q * (1.0 / D) - mean * mean) * (D / (D - 1.0))
                    var = jnp.maximum(var, 1e-30)
                    std = var * _rsqrt(var)
                    rinv = _recip(std + 1e-6)
                    for k in range(NK):
                        o = (h[k] - mean) * rinv
                        out_buf[ofr, pl.ds((j & 1) * D + k * L, L)] = \
                            o * wv[k] + bv[k]

        gather(0, fidx_a, rows_a, sem_ga).start()

        def do_pair(j, _):
            c0 = 2 * j
            c1 = c0 + 1
            gather(c1, fidx_b, rows_b, sem_gb).start()
            wait_gather(c0, fidx_a, rows_a, sem_ga).wait()

            @pl.when(j > 0)
            def _():
                store(j - 1, sem_s).wait()

            compute(c0, rows_a, out_p, 0)

            @pl.when(j < seqs_per_w // 2 - 1)
            def _():
                gather(c0 + 2, fidx_a, rows_a, sem_ga).start()

            wait_gather(c1, fidx_b, rows_b, sem_gb).wait()
            compute(c1, rows_b, out_p, SEQ // 2)
            store(j, sem_s).start()
            return 0

        last = seqs_per_w // 2 - 1
        lax.fori_loop(0, seqs_per_w // 2, do_pair, 0)
        store(last, sem_s).wait()

    return pl.kernel(
        body,
        out_type=jax.ShapeDtypeStruct((n_rows // 2, 2 * D), jnp.float32),
        mesh=mesh,
        compiler_params=pltpu.CompilerParams(
            needs_layout_passes=False, use_tc_tiling_on_sc=True),
        scratch_types=[
            pltpu.VMEM((rows_per_w + L,), jnp.int32),
            pltpu.VMEM((SEQ // 2, 2 * D), jnp.float32),
            pltpu.VMEM((D,), jnp.float32),
            pltpu.VMEM((D,), jnp.float32),
            pltpu.VMEM((SEQ, 2 * D), jnp.float32),
            pltpu.VMEM((SEQ, 2 * D), jnp.float32),
            pltpu.VMEM((SEQ, 2 * D), jnp.float32),
            pltpu.VMEM((NG * L,), jnp.int32),
            pltpu.VMEM((NG * L,), jnp.int32),
            pltpu.SemaphoreType.DMA,
            pltpu.SemaphoreType.DMA,
            pltpu.SemaphoreType.DMA,
        ],
    )


def kernel(x, table, ln_weight, ln_bias):
    batch, seq = x.shape
    d_model = table.shape[1]
    assert seq == SEQ and d_model == D
    n_rows = batch * seq

    pe = _positional_encoding(d_model, seq).reshape(seq // 2, 2 * d_model)
    tablef = table.astype(jnp.float32).reshape(-1, 2 * d_model)
    x_flat = x.reshape(n_rows).astype(jnp.int32)

    out = _make_sc_call(n_rows)(
        x_flat, tablef, pe,
        ln_weight.astype(jnp.float32), ln_bias.astype(jnp.float32))
    return out.reshape(batch, seq, d_model)


# speedup vs baseline: 1.9508x; 1.9508x over previous
import math

import jax
import jax.numpy as jnp
from jax import lax
from jax.experimental import pallas as pl
from jax.experimental.pallas import tpu as pltpu
from jax.experimental.pallas import tpu_sc as plsc

D = 64
SEQ = 200
NW = 32
L = 16
NK = D // L
SCALE = math.sqrt(D)


def _positional_encoding(d_model: int, length: int) -> jax.Array:
    position = jnp.arange(length, dtype=jnp.float32)[:, None]
    div_term = jnp.exp(
        jnp.arange(0, d_model, 2, dtype=jnp.float32)
        * (-(math.log(10000.0) / d_model))
    )
    pe = jnp.zeros((length, d_model), dtype=jnp.float32)
    pe = pe.at[:, 0::2].set(jnp.sin(position * div_term))
    pe = pe.at[:, 1::2].set(jnp.cos(position * div_term))
    return pe


def _rsqrt(v):
    i = lax.bitcast_convert_type(v, jnp.int32)
    i = 0x5F3759DF - lax.shift_right_logical(i, 1)
    y = lax.bitcast_convert_type(i, jnp.float32)
    for _ in range(2):
        y = y * (1.5 - 0.5 * v * y * y)
    return y


def _recip(v):
    i = lax.bitcast_convert_type(v, jnp.int32)
    i = 0x7EF311C3 - i
    y = lax.bitcast_convert_type(i, jnp.float32)
    for _ in range(2):
        y = y * (2.0 - v * y)
    return y


def _make_sc_call(n_rows: int):
    assert n_rows % (NW * SEQ) == 0
    rows_per_w = n_rows // NW
    seqs_per_w = rows_per_w // SEQ

    mesh = plsc.VectorSubcoreMesh(core_axis_name="c", subcore_axis_name="s")

    def body(x_hbm, table_hbm, pe_hbm, w_hbm, b_hbm, out_hbm,
             idx_v, pe_v, w_v, b_v, rows_a, rows_b, out_a, out_b,
             sem_ga, sem_gb, sem_sa, sem_sb):
        cid = lax.axis_index("c")
        sid = lax.axis_index("s")
        wid = sid * 2 + cid
        base = wid * rows_per_w

        pltpu.sync_copy(x_hbm.at[pl.ds(base, rows_per_w)], idx_v)
        pltpu.sync_copy(pe_hbm, pe_v)
        pltpu.sync_copy(w_hbm, w_v)
        pltpu.sync_copy(b_hbm, b_v)

        wv = [w_v[pl.ds(k * L, L)] for k in range(NK)]
        bv = [b_v[pl.ds(k * L, L)] for k in range(NK)]

        def gather(c, rows_buf, sem):
            idx_ref = idx_v.at[pl.ds(c * SEQ, SEQ)]
            return pltpu.make_async_copy(
                table_hbm.at[idx_ref], rows_buf, sem)

        def store(c, out_buf, sem):
            return pltpu.make_async_copy(
                out_buf, out_hbm.at[pl.ds((base + c * SEQ) * D, SEQ * D)], sem)

        def compute(rows_buf, out_buf):
            @plsc.parallel_loop(0, SEQ, unroll=8)
            def row_body(r):
                h = []
                for k in range(NK):
                    e = rows_buf[r, pl.ds(k * L, L)]
                    p = pe_v[r, pl.ds(k * L, L)]
                    h.append(e * SCALE + p)
                t = (h[0] + h[1]) + (h[2] + h[3])
                s = jnp.sum(t)
                qv = (h[0] * h[0] + h[1] * h[1]) + (h[2] * h[2] + h[3] * h[3])
                q = jnp.sum(qv)
                mean = s * (1.0 / D)
                var = (q * (1.0 / D) - mean * mean) * (D / (D - 1.0))
                var = jnp.maximum(var, 1e-30)
                std = var * _rsqrt(var)
                rinv = _recip(std + 1e-6)
                for k in range(NK):
                    o = (h[k] - mean) * rinv
                    out_buf[pl.ds(r * D + k * L, L)] = o * wv[k] + bv[k]

        gather(0, rows_a, sem_ga).start()

        def do_pair(j, _):
            c0 = 2 * j
            c1 = c0 + 1
            gather(c1, rows_b, sem_gb).start()
            gather(c0, rows_a, sem_ga).wait()

            @pl.when(j > 0)
            def _():
                store(c0 - 2, out_a, sem_sa).wait()

            compute(rows_a, out_a)
            store(c0, out_a, sem_sa).start()

            @pl.when(j < seqs_per_w // 2 - 1)
            def _():
                gather(c0 + 2, rows_a, sem_ga).start()

            gather(c1, rows_b, sem_gb).wait()

            @pl.when(j > 0)
            def _():
                store(c1 - 2, out_b, sem_sb).wait()

            compute(rows_b, out_b)
            store(c1, out_b, sem_sb).start()
            return 0

        last = seqs_per_w // 2 - 1
        lax.fori_loop(0, seqs_per_w // 2, do_pair, 0)
        store(2 * last, out_a, sem_sa).wait()
        store(2 * last + 1, out_b, sem_sb).wait()

    return pl.kernel(
        body,
        out_type=jax.ShapeDtypeStruct((n_rows * D,), jnp.float32),
        mesh=mesh,
        compiler_params=pltpu.CompilerParams(
            needs_layout_passes=False, use_tc_tiling_on_sc=False),
        scratch_types=[
            pltpu.VMEM((rows_per_w,), jnp.int32),
            pltpu.VMEM((SEQ, D), jnp.float32),
            pltpu.VMEM((D,), jnp.float32),
            pltpu.VMEM((D,), jnp.float32),
            pltpu.VMEM((SEQ, D), jnp.float32),
            pltpu.VMEM((SEQ, D), jnp.float32),
            pltpu.VMEM((SEQ * D,), jnp.float32),
            pltpu.VMEM((SEQ * D,), jnp.float32),
            pltpu.SemaphoreType.DMA,
            pltpu.SemaphoreType.DMA,
            pltpu.SemaphoreType.DMA,
            pltpu.SemaphoreType.DMA,
        ],
    )


def kernel(x, table, ln_weight, ln_bias):
    batch, seq = x.shape
    d_model = table.shape[1]
    assert seq == SEQ and d_model == D
    n_rows = batch * seq

    pe = _positional_encoding(d_model, seq)
    x_flat = x.reshape(n_rows).astype(jnp.int32)

    out = _make_sc_call(n_rows)(
        x_flat, table.astype(jnp.float32), pe,
        ln_weight.astype(jnp.float32), ln_bias.astype(jnp.float32))
    return out.reshape(batch, seq, d_model)
